# baseline (speedup 1.0000x reference)
import functools

import jax
import jax.numpy as jnp
from jax import lax
from jax.experimental import pallas as pl
from jax.experimental.pallas import tpu as pltpu
from jax.experimental.pallas import tpu_sc as plsc

_LANES = 384
_ROWS = 147456

_BLOCK_ROWS = 16384
_CH = 8
_UNROLL = 8
_TC_BLOCKS = 7

_NC = 2
_NS = 16
_NW = _NC * _NS
_L = 16
_BUF_ROWS = 128


def _tc_body(x_ref, out_ref, acc_o, acc_u):
    i = pl.program_id(0)

    def body(k, carry):
        ao, au = carry
        base = k * (_CH * _UNROLL)
        for j in range(_UNROLL):
            x = x_ref[pl.ds(base + j * _CH, _CH), :]
            t = x - 1.0
            o = jnp.maximum(t, 0.0)
            u = jnp.minimum(x, 0.0)
            ao = ao + o * o
            au = au + u * u
        return ao, au

    z = jnp.zeros((_CH, _LANES), jnp.float32)
    n_iter = _BLOCK_ROWS // (_CH * _UNROLL)
    ao, au = lax.fori_loop(0, n_iter, body, (z, z))

    @pl.when(i == 0)
    def _init():
        acc_o[...] = jnp.zeros_like(acc_o)
        acc_u[...] = jnp.zeros_like(acc_u)

    acc_o[...] += ao
    acc_u[...] += au

    @pl.when(i == pl.num_programs(0) - 1)
    def _fini():
        out_ref[0, 0] = jnp.sum(acc_o[...])
        out_ref[0, 1] = jnp.sum(acc_u[...])


def _tc_partials(x2d, n_blocks):
    return pl.pallas_call(
        _tc_body,
        grid=(n_blocks,),
        in_specs=[pl.BlockSpec((_BLOCK_ROWS, _LANES), lambda i: (i, 0))],
        out_specs=pl.BlockSpec(
            (1, 2), lambda i: (0, 0), memory_space=pltpu.SMEM
        ),
        out_shape=jax.ShapeDtypeStruct((1, 2), jnp.float32),
        scratch_shapes=[
            pltpu.VMEM((_CH, _LANES), jnp.float32),
            pltpu.VMEM((_CH, _LANES), jnp.float32),
        ],
        compiler_params=pltpu.CompilerParams(
            dimension_semantics=("arbitrary",),
        ),
    )(x2d)


def _sc_accumulate(buf, carry_in):
    def body(r, carry):
        a0, b0, a1, b1 = carry
        for c in range(_LANES // _L):
            x = buf[r, pl.ds(c * _L, _L)]
            t = x - 1.0
            o = jnp.maximum(t, 0.0)
            u = jnp.minimum(x, 0.0)
            if c % 2 == 0:
                a0 = a0 + o * o
                b0 = b0 + u * u
            else:
                a1 = a1 + o * o
                b1 = b1 + u * u
        return a0, b0, a1, b1

    return lax.fori_loop(0, _BUF_ROWS, body, carry_in)


def _sc_partials(x2d, start_row, n_rows):
    chunk = n_rows // _NS
    npieces = chunk // _BUF_ROWS
    assert n_rows % _NS == 0 and chunk % _BUF_ROWS == 0 and npieces % 2 == 0

    mesh = plsc.VectorSubcoreMesh(
        core_axis_name="c", subcore_axis_name="s", num_cores=1)

    @functools.partial(
        pl.kernel,
        mesh=mesh,
        out_type=(
            jax.ShapeDtypeStruct((_NS, _L), jnp.float32),
            jax.ShapeDtypeStruct((_NS, _L), jnp.float32),
        ),
        scratch_types=[
            pltpu.VMEM((_BUF_ROWS, _LANES), jnp.float32),
            pltpu.VMEM((_BUF_ROWS, _LANES), jnp.float32),
            pltpu.VMEM((_L,), jnp.float32),
            pltpu.VMEM((_L,), jnp.float32),
            pltpu.SemaphoreType.DMA,
            pltpu.SemaphoreType.DMA,
        ],
        compiler_params=pltpu.CompilerParams(use_tc_tiling_on_sc=True),
    )
    def sc_kernel(x_hbm, out_o, out_u, buf0, buf1, vo, vu, sem0, sem1):
        wid = lax.axis_index("s")
        base = start_row + wid * chunk
        last = npieces - 1

        pltpu.async_copy(x_hbm.at[pl.ds(base, _BUF_ROWS)], buf0, sem0)
        pltpu.async_copy(
            x_hbm.at[pl.ds(base + _BUF_ROWS, _BUF_ROWS)], buf1, sem1)

        def outer(q, carry):
            pltpu.make_async_copy(
                x_hbm.at[pl.ds(base, _BUF_ROWS)], buf0, sem0).wait()
            carry = _sc_accumulate(buf0, carry)
            p0 = jnp.minimum(2 * q + 2, last) * _BUF_ROWS
            pltpu.async_copy(x_hbm.at[pl.ds(base + p0, _BUF_ROWS)], buf0, sem0)

            pltpu.make_async_copy(
                x_hbm.at[pl.ds(base, _BUF_ROWS)], buf1, sem1).wait()
            carry = _sc_accumulate(buf1, carry)
            p1 = jnp.minimum(2 * q + 3, last) * _BUF_ROWS
            pltpu.async_copy(x_hbm.at[pl.ds(base + p1, _BUF_ROWS)], buf1, sem1)
            return carry

        z = jnp.zeros((_L,), jnp.float32)
        ao0, au0, ao1, au1 = lax.fori_loop(
            0, npieces // 2, outer, (z, z, z, z))

        pltpu.make_async_copy(
            x_hbm.at[pl.ds(base, _BUF_ROWS)], buf0, sem0).wait()
        pltpu.make_async_copy(
            x_hbm.at[pl.ds(base, _BUF_ROWS)], buf1, sem1).wait()

        vo[...] = ao0 + ao1
        vu[...] = au0 + au1
        pltpu.sync_copy(vo, out_o.at[wid])
        pltpu.sync_copy(vu, out_u.at[wid])

    return sc_kernel(x2d)


def kernel(tensor):
    x2d = tensor.reshape(_ROWS, _LANES)
    tc_rows = _TC_BLOCKS * _BLOCK_ROWS

    sc_rows = _ROWS - tc_rows
    half = sc_rows // 2
    sc_o0, sc_u0 = _sc_partials(x2d, tc_rows, half)
    sc_o1, sc_u1 = _sc_partials(x2d, tc_rows + half, sc_rows - half)
    tc = _tc_partials(x2d, _TC_BLOCKS)

    s_o = tc[0, 0] + jnp.sum(sc_o0) + jnp.sum(sc_o1)
    s_u = tc[0, 1] + jnp.sum(sc_u0) + jnp.sum(sc_u1)
    return jnp.sqrt(s_o) + jnp.sqrt(s_u)


# speedup vs baseline: 1.1336x; 1.1336x over previous
import functools

import jax
import jax.numpy as jnp
from jax import lax
from jax.experimental import pallas as pl
from jax.experimental.pallas import tpu as pltpu
from jax.experimental.pallas import tpu_sc as plsc

_LANES = 384
_ROWS = 147456

_BLOCK_ROWS = 8192
_CH = 8
_UNROLL = 8
_TC_BLOCKS = 16

_NC = 2
_NS = 16
_NW = _NC * _NS
_L = 16
_BUF_ROWS = 128


def _tc_body(x_ref, out_ref, acc_o, acc_u):
    i = pl.program_id(0)

    def body(k, carry):
        ao, au = carry
        base = k * (_CH * _UNROLL)
        for j in range(_UNROLL):
            x = x_ref[pl.ds(base + j * _CH, _CH), :]
            t = x - 1.0
            o = jnp.maximum(t, 0.0)
            u = jnp.minimum(x, 0.0)
            ao = ao + o * o
            au = au + u * u
        return ao, au

    z = jnp.zeros((_CH, _LANES), jnp.float32)
    n_iter = _BLOCK_ROWS // (_CH * _UNROLL)
    ao, au = lax.fori_loop(0, n_iter, body, (z, z))

    @pl.when(i == 0)
    def _init():
        acc_o[...] = jnp.zeros_like(acc_o)
        acc_u[...] = jnp.zeros_like(acc_u)

    acc_o[...] += ao
    acc_u[...] += au

    @pl.when(i == pl.num_programs(0) - 1)
    def _fini():
        out_ref[0, 0] = jnp.sum(acc_o[...])
        out_ref[0, 1] = jnp.sum(acc_u[...])


def _tc_partials(x2d, n_blocks):
    return pl.pallas_call(
        _tc_body,
        grid=(n_blocks,),
        in_specs=[pl.BlockSpec((_BLOCK_ROWS, _LANES), lambda i: (i, 0))],
        out_specs=pl.BlockSpec(
            (1, 2), lambda i: (0, 0), memory_space=pltpu.SMEM
        ),
        out_shape=jax.ShapeDtypeStruct((1, 2), jnp.float32),
        scratch_shapes=[
            pltpu.VMEM((_CH, _LANES), jnp.float32),
            pltpu.VMEM((_CH, _LANES), jnp.float32),
        ],
        compiler_params=pltpu.CompilerParams(
            dimension_semantics=("arbitrary",),
        ),
    )(x2d)


def _sc_accumulate(buf, carry_in):
    def body(r, carry):
        a0, b0, a1, b1 = carry
        for c in range(_LANES // _L):
            x = buf[r, pl.ds(c * _L, _L)]
            t = x - 1.0
            o = jnp.maximum(t, 0.0)
            u = jnp.minimum(x, 0.0)
            if c % 2 == 0:
                a0 = a0 + o * o
                b0 = b0 + u * u
            else:
                a1 = a1 + o * o
                b1 = b1 + u * u
        return a0, b0, a1, b1

    return lax.fori_loop(0, _BUF_ROWS, body, carry_in)


def _sc_partials(x2d, start_row, n_rows):
    chunk = n_rows // _NW
    npieces = chunk // _BUF_ROWS
    assert n_rows % _NW == 0 and chunk % _BUF_ROWS == 0 and npieces % 2 == 0

    mesh = plsc.VectorSubcoreMesh(core_axis_name="c", subcore_axis_name="s")

    @functools.partial(
        pl.kernel,
        mesh=mesh,
        out_type=(
            jax.ShapeDtypeStruct((_NW, _L), jnp.float32),
            jax.ShapeDtypeStruct((_NW, _L), jnp.float32),
        ),
        scratch_types=[
            pltpu.VMEM((_BUF_ROWS, _LANES), jnp.float32),
            pltpu.VMEM((_BUF_ROWS, _LANES), jnp.float32),
            pltpu.VMEM((_L,), jnp.float32),
            pltpu.VMEM((_L,), jnp.float32),
            pltpu.SemaphoreType.DMA,
            pltpu.SemaphoreType.DMA,
        ],
        compiler_params=pltpu.CompilerParams(use_tc_tiling_on_sc=True),
    )
    def sc_kernel(x_hbm, out_o, out_u, buf0, buf1, vo, vu, sem0, sem1):
        wid = lax.axis_index("s") * _NC + lax.axis_index("c")
        base = start_row + wid * chunk
        last = npieces - 1

        pltpu.async_copy(x_hbm.at[pl.ds(base, _BUF_ROWS)], buf0, sem0)
        pltpu.async_copy(
            x_hbm.at[pl.ds(base + _BUF_ROWS, _BUF_ROWS)], buf1, sem1)

        def outer(q, carry):
            pltpu.make_async_copy(
                x_hbm.at[pl.ds(base, _BUF_ROWS)], buf0, sem0).wait()
            carry = _sc_accumulate(buf0, carry)
            p0 = jnp.minimum(2 * q + 2, last) * _BUF_ROWS
            pltpu.async_copy(x_hbm.at[pl.ds(base + p0, _BUF_ROWS)], buf0, sem0)

            pltpu.make_async_copy(
                x_hbm.at[pl.ds(base, _BUF_ROWS)], buf1, sem1).wait()
            carry = _sc_accumulate(buf1, carry)
            p1 = jnp.minimum(2 * q + 3, last) * _BUF_ROWS
            pltpu.async_copy(x_hbm.at[pl.ds(base + p1, _BUF_ROWS)], buf1, sem1)
            return carry

        z = jnp.zeros((_L,), jnp.float32)
        ao0, au0, ao1, au1 = lax.fori_loop(
            0, npieces // 2, outer, (z, z, z, z))

        pltpu.make_async_copy(
            x_hbm.at[pl.ds(base, _BUF_ROWS)], buf0, sem0).wait()
        pltpu.make_async_copy(
            x_hbm.at[pl.ds(base, _BUF_ROWS)], buf1, sem1).wait()

        vo[...] = ao0 + ao1
        vu[...] = au0 + au1
        pltpu.sync_copy(vo, out_o.at[wid])
        pltpu.sync_copy(vu, out_u.at[wid])

    return sc_kernel(x2d)


def kernel(tensor):
    x2d = tensor.reshape(_ROWS, _LANES)
    tc_rows = _TC_BLOCKS * _BLOCK_ROWS

    sc_o, sc_u = _sc_partials(x2d, tc_rows, _ROWS - tc_rows)
    tc = _tc_partials(x2d, _TC_BLOCKS)

    s_o = tc[0, 0] + jnp.sum(sc_o)
    s_u = tc[0, 1] + jnp.sum(sc_u)
    return jnp.sqrt(s_o) + jnp.sqrt(s_u)


# speedup vs baseline: 1.5062x; 1.3286x over previous
import jax
import jax.numpy as jnp
from jax.experimental import pallas as pl
from jax.experimental.pallas import tpu as pltpu

_LANES = 384
_BLOCK_ROWS = 16384
_CH = 8
_UNROLL = 8


def _reduce_body(x_ref, out_ref, acc_o, acc_u):
    i = pl.program_id(0)

    def body(k, carry):
        ao, au = carry
        base = k * (_CH * _UNROLL)
        for j in range(_UNROLL):
            x = x_ref[pl.ds(base + j * _CH, _CH), :]
            t = x - 1.0
            o = jnp.maximum(t, 0.0)
            u = jnp.minimum(x, 0.0)
            ao = ao + o * o
            au = au + u * u
        return ao, au

    z = jnp.zeros((_CH, _LANES), jnp.float32)
    n_iter = _BLOCK_ROWS // (_CH * _UNROLL)
    ao, au = jax.lax.fori_loop(0, n_iter, body, (z, z))

    @pl.when(i == 0)
    def _init():
        acc_o[...] = jnp.zeros_like(acc_o)
        acc_u[...] = jnp.zeros_like(acc_u)

    acc_o[...] += ao
    acc_u[...] += au

    @pl.when(i == pl.num_programs(0) - 1)
    def _fini():
        s_o = jnp.sum(acc_o[...])
        s_u = jnp.sum(acc_u[...])
        out_ref[0, 0] = jnp.sqrt(s_o) + jnp.sqrt(s_u)


def kernel(tensor):
    n = tensor.size
    rows = n // _LANES
    x2d = tensor.reshape(rows, _LANES)
    grid = rows // _BLOCK_ROWS

    out = pl.pallas_call(
        _reduce_body,
        grid=(grid,),
        in_specs=[pl.BlockSpec((_BLOCK_ROWS, _LANES), lambda i: (i, 0))],
        out_specs=pl.BlockSpec(
            (1, 1), lambda i: (0, 0), memory_space=pltpu.SMEM
        ),
        out_shape=jax.ShapeDtypeStruct((1, 1), jnp.float32),
        scratch_shapes=[
            pltpu.VMEM((_CH, _LANES), jnp.float32),
            pltpu.VMEM((_CH, _LANES), jnp.float32),
        ],
        compiler_params=pltpu.CompilerParams(
            dimension_semantics=("arbitrary",),
        ),
    )(x2d)
    return out[0, 0]
